# baseline (speedup 1.0000x reference)
import functools

import jax
import jax.numpy as jnp
from jax import lax
from jax.experimental import pallas as pl
from jax.experimental.pallas import tpu as pltpu
from jax.experimental.pallas import tpu_sc as plsc

NC = 2
NS = 16
NW = NC * NS
L = 16
ROW = 32
P = 2048
B_SC = 6
HR = 8


def _sc_partial_hist(inp, tgt, C):
    _, _, H, W = inp.shape
    HW = H * W
    ppw = HW // NW
    nchunk = ppw // P
    total = B_SC * nchunk
    RPC = P // W
    rpt = ppw // W

    mesh = plsc.VectorSubcoreMesh(core_axis_name="c", subcore_axis_name="s")

    @functools.partial(
        pl.kernel,
        mesh=mesh,
        compiler_params=pltpu.CompilerParams(needs_layout_passes=False),
        out_type=jax.ShapeDtypeStruct((NW, L * C * ROW), jnp.int32),
        scratch_types=[
            pltpu.VMEM((2, C, RPC, W), jnp.float32),
            pltpu.VMEM((2, RPC, W), jnp.int32),
            pltpu.VMEM((L * C * ROW,), jnp.int32),
            pltpu.SemaphoreType.DMA,
            pltpu.SemaphoreType.DMA,
        ],
    )
    def k(in_hbm, tg_hbm, out_hbm, xbufs, tbufs, hist, sem0, sem1):
        wid = lax.axis_index("s") * NC + lax.axis_index("c")
        sems = (sem0, sem1)
        HB = C * ROW
        lane_off = lax.broadcasted_iota(jnp.int32, (L,), 0) * HB
        zeros = jnp.zeros((L,), jnp.int32)
        ones = jnp.ones((L,), jnp.int32)

        def zero_body(i, _):
            hist[pl.ds(i * L, L)] = zeros
            return 0

        lax.fori_loop(0, (L * HB) // L, zero_body, 0)

        def issue(ci, slot):
            b = ci // nchunk
            r0 = wid * rpt + (ci % nchunk) * RPC
            pltpu.async_copy(in_hbm.at[b, :, pl.ds(r0, RPC), :],
                             xbufs.at[slot], sems[slot])
            pltpu.async_copy(tg_hbm.at[b, pl.ds(r0, RPC), :],
                             tbufs.at[slot], sems[slot])

        def wait(slot):
            pltpu.make_async_copy(in_hbm.at[0, :, pl.ds(0, RPC), :],
                                  xbufs.at[slot], sems[slot]).wait()
            pltpu.make_async_copy(tg_hbm.at[0, pl.ds(0, RPC), :],
                                  tbufs.at[slot], sems[slot]).wait()

        def group(slot, r, col):
            items = [(xbufs[slot, c, r, pl.ds(col, L)], c) for c in range(C)]
            while len(items) > 1:
                nxt = []
                for j in range(0, len(items) - 1, 2):
                    pm, pa = items[j]
                    qm, qa = items[j + 1]
                    gt = qm > pm
                    nxt.append((jnp.where(gt, qm, pm), jnp.where(gt, qa, pa)))
                if len(items) % 2:
                    nxt.append(items[-1])
                items = nxt
            a = items[0][1]
            t = tbufs[slot, r, pl.ds(col, L)]
            addr = lane_off + t * ROW + a
            plsc.addupdate_scatter(hist, [addr], ones)

        def compute(slot):
            for r in range(RPC):
                @plsc.parallel_loop(0, W // L, unroll=4)
                def _(i):
                    group(slot, r, i * L)

        issue(0, 0)
        issue(1, 1)

        def pair_body(cp, _):
            ci = cp * 2
            wait(0)
            compute(0)

            @pl.when(ci + 2 < total)
            def _():
                issue(ci + 2, 0)

            wait(1)
            compute(1)

            @pl.when(ci + 3 < total)
            def _():
                issue(ci + 3, 1)

            return 0

        lax.fori_loop(0, total // 2, pair_body, 0)
        pltpu.sync_copy(hist, out_hbm.at[wid])

    return k(inp, tgt)


def _tc_partial(inp, tgt, C):
    B, _, H, W = inp.shape
    bpb = H // HR
    nb = (B - B_SC) * bpb

    def body(x_ref, t_ref, o_ref):
        @pl.when(pl.program_id(0) == 0)
        def _():
            o_ref[...] = jnp.zeros_like(o_ref)

        iota_c = lax.broadcasted_iota(jnp.int32, (C, W), 0)
        acc = jnp.zeros((C, C), jnp.float32)
        for r in range(HR):
            xr = x_ref[0, :, r, :]
            m = jnp.max(xr, axis=0)
            eq = xr == m[None, :]
            pred = jnp.min(jnp.where(eq, iota_c, C), axis=0)
            tr = t_ref[0, r, :]
            oh_t = (iota_c == tr[None, :]).astype(jnp.float32)
            oh_p = (iota_c == pred[None, :]).astype(jnp.float32)
            acc += lax.dot_general(oh_t, oh_p, (((1,), (1,)), ((), ())),
                                   preferred_element_type=jnp.float32)
        o_ref[...] += acc

    return pl.pallas_call(
        body,
        grid=(nb,),
        in_specs=[
            pl.BlockSpec((1, C, HR, W),
                         lambda i: (B_SC + i // bpb, 0, i % bpb, 0)),
            pl.BlockSpec((1, HR, W),
                         lambda i: (B_SC + i // bpb, i % bpb, 0)),
        ],
        out_specs=pl.BlockSpec((C, C), lambda i: (0, 0)),
        out_shape=jax.ShapeDtypeStruct((C, C), jnp.float32),
    )(inp, tgt)


def _merge(parts, tc_conf, C):
    def body(x_ref, y_ref, o_ref):
        o_ref[...] = (jnp.sum(x_ref[...], axis=(0, 1))[:, :C]
                      + y_ref[...].astype(jnp.int32))

    return pl.pallas_call(
        body,
        out_shape=jax.ShapeDtypeStruct((C, C), jnp.int32),
    )(parts, tc_conf)


def kernel(input, target, class_num):
    C = input.shape[1]
    sc_parts = _sc_partial_hist(input, target, C)
    tc_conf = _tc_partial(input, target, C)
    sc_parts = sc_parts.reshape(NW, L, C, ROW)
    return _merge(sc_parts, tc_conf, C)
